# baseline (speedup 1.0000x reference)
import jax
import jax.numpy as jnp
from jax.experimental import pallas as pl

EPS = 1e-5
K = 16


def _bn(x, gamma, beta, axes):
    m = jnp.mean(x, axis=axes, keepdims=True)
    v = jnp.var(x, axis=axes, keepdims=True)
    return gamma * (x - m) / jnp.sqrt(v + EPS) + beta


def _mlp(x, W, b, gamma=None, beta=None, act='relu', axes=(0, 1, 2)):
    y = jnp.einsum('...i,oi->...o', x, W) + b
    if gamma is not None:
        y = _bn(y, gamma, beta, axes)
    if act == 'relu':
        y = jax.nn.relu(y)
    elif act == 'lrelu':
        y = jnp.where(y > 0, y, 0.2 * y)
    return y


def _knn_idx(p, k):
    sq = jnp.sum(p * p, axis=-1)
    d2 = sq[:, :, None] + sq[:, None, :] - 2.0 * jnp.einsum('bic,bjc->bij', p, p)
    d = jnp.sqrt(jnp.maximum(d2, 0.0))
    _, idx = jax.lax.top_k(d, k)
    return idx


def _gather(points, idx):
    return jax.vmap(lambda pts, ix: pts[ix])(points, idx)


def kernel(p, f, params):
    idx = _knn_idx(p, K)
    f0 = _mlp(f, params['W0'], params['b0'], params['g0'], params['e0'], 'relu', (0, 1))
    p_knn = _gather(p, idx)
    f_knn = _gather(f0, idx)
    def embed(x, xk):
        xe = jnp.broadcast_to(x[:, :, None, :], xk.shape)
        return jnp.concatenate([xe, xk - xe], axis=-1)
    lgc = embed(p, p_knn)
    lsc = embed(f0, f_knn)
    p_off = _mlp(lsc, params['W1'], params['b1'], params['g1'], params['e1'], 'relu')
    p_tilde = p_off + p_knn
    lgc_aug = jnp.concatenate([lgc, p_tilde], axis=-1)
    f_off = _mlp(lgc_aug, params['W2'], params['b2'], params['g2'], params['e2'], 'relu')
    f_tilde = f_off + f_knn
    lsc_aug = jnp.concatenate([lsc, f_tilde], axis=-1)
    p_enc = _mlp(lgc_aug, params['W3'], params['b3'], params['g3'], params['e3'], 'relu')
    f_enc = _mlp(lsc_aug, params['W4'], params['b4'], params['g4'], params['e4'], 'relu')
    alc = jnp.concatenate([p_enc, f_enc], axis=-1)
    kw = jnp.einsum('...i,oi->...o', alc, params['Wm0']) + params['bm0']
    kw = jax.nn.softmax(kw, axis=2)
    ws = jnp.sum(alc * kw, axis=2, keepdims=True)
    mx = jnp.max(alc, axis=2, keepdims=True)
    mla = jnp.concatenate([ws, mx], axis=-1)
    mla = _mlp(mla, params['Wm1'], params['bm1'], params['gm1'], params['em1'], 'relu')
    mla = _mlp(mla, params['Wm2'], params['bm2'], params['gm2'], params['em2'], 'lrelu')
    return jnp.squeeze(mla, axis=2), p_tilde


# speedup vs baseline: 2.2335x; 2.2335x over previous
import jax
import jax.numpy as jnp
from jax.experimental import pallas as pl

EPS = 1e-5
K = 16
_NEG = -jnp.inf


def _bn(x, gamma, beta, axes):
    m = jnp.mean(x, axis=axes, keepdims=True)
    v = jnp.var(x, axis=axes, keepdims=True)
    return gamma * (x - m) / jnp.sqrt(v + EPS) + beta


def _mlp(x, W, b, gamma=None, beta=None, act='relu', axes=(0, 1, 2)):
    y = jnp.einsum('...i,oi->...o', x, W) + b
    if gamma is not None:
        y = _bn(y, gamma, beta, axes)
    if act == 'relu':
        y = jax.nn.relu(y)
    elif act == 'lrelu':
        y = jnp.where(y > 0, y, 0.2 * y)
    return y


def _knn_body(pr_ref, pa_ref, sqa_ref, idx_ref):
    R = pr_ref.shape[1]
    N = pa_ref.shape[1]
    pr = pr_ref[0]
    pa = pa_ref[0]
    sqa = sqa_ref[0]
    sqr = (pr[:, 0] * pr[:, 0] + pr[:, 1] * pr[:, 1]) + pr[:, 2] * pr[:, 2]
    dot = jax.lax.dot_general(pr.astype(jnp.bfloat16), pa.astype(jnp.bfloat16),
                              (((1,), (1,)), ((), ())),
                              preferred_element_type=jnp.float32)
    d2 = sqr[:, None] + sqa - 2.0 * dot
    d = jnp.sqrt(jnp.maximum(d2, 0.0))
    iota = jax.lax.broadcasted_iota(jnp.int32, (R, N), 1)
    cols = []
    for _ in range(K):
        m = jnp.max(d, axis=1, keepdims=True)
        sel = jnp.where(d == m, iota, N)
        j = jnp.min(sel, axis=1, keepdims=True)
        cols.append(j)
        d = jnp.where(iota == j, _NEG, d)
    idx_ref[0] = jnp.concatenate(cols, axis=1)


def _knn_idx_pallas(p):
    B, N, _ = p.shape
    R = 512
    sq = jnp.sum(p * p, axis=-1)
    return pl.pallas_call(
        _knn_body,
        grid=(B, N // R),
        in_specs=[pl.BlockSpec((1, R, 3), lambda b, r: (b, r, 0)),
                  pl.BlockSpec((1, N, 3), lambda b, r: (b, 0, 0)),
                  pl.BlockSpec((1, 1, N), lambda b, r: (b, 0, 0))],
        out_specs=pl.BlockSpec((1, R, K), lambda b, r: (b, r, 0)),
        out_shape=jax.ShapeDtypeStruct((B, N, K), jnp.int32),
    )(p, p, sq[:, None, :])


def _gather(points, idx):
    return jax.vmap(lambda pts, ix: pts[ix])(points, idx)


def kernel(p, f, params):
    idx = _knn_idx_pallas(p)
    f0 = _mlp(f, params['W0'], params['b0'], params['g0'], params['e0'], 'relu', (0, 1))
    p_knn = _gather(p, idx)
    f_knn = _gather(f0, idx)
    def embed(x, xk):
        xe = jnp.broadcast_to(x[:, :, None, :], xk.shape)
        return jnp.concatenate([xe, xk - xe], axis=-1)
    lgc = embed(p, p_knn)
    lsc = embed(f0, f_knn)
    p_off = _mlp(lsc, params['W1'], params['b1'], params['g1'], params['e1'], 'relu')
    p_tilde = p_off + p_knn
    lgc_aug = jnp.concatenate([lgc, p_tilde], axis=-1)
    f_off = _mlp(lgc_aug, params['W2'], params['b2'], params['g2'], params['e2'], 'relu')
    f_tilde = f_off + f_knn
    lsc_aug = jnp.concatenate([lsc, f_tilde], axis=-1)
    p_enc = _mlp(lgc_aug, params['W3'], params['b3'], params['g3'], params['e3'], 'relu')
    f_enc = _mlp(lsc_aug, params['W4'], params['b4'], params['g4'], params['e4'], 'relu')
    alc = jnp.concatenate([p_enc, f_enc], axis=-1)
    kw = jnp.einsum('...i,oi->...o', alc, params['Wm0']) + params['bm0']
    kw = jax.nn.softmax(kw, axis=2)
    ws = jnp.sum(alc * kw, axis=2, keepdims=True)
    mx = jnp.max(alc, axis=2, keepdims=True)
    mla = jnp.concatenate([ws, mx], axis=-1)
    mla = _mlp(mla, params['Wm1'], params['bm1'], params['gm1'], params['em1'], 'relu')
    mla = _mlp(mla, params['Wm2'], params['bm2'], params['gm2'], params['em2'], 'lrelu')
    return jnp.squeeze(mla, axis=2), p_tilde


# speedup vs baseline: 6.8438x; 3.0642x over previous
import jax
import jax.numpy as jnp
from jax.experimental import pallas as pl

EPS = 1e-5
K = 16
_NEG = -jnp.inf


def _knn_body(pr_ref, pa_ref, sqa_ref, idx_ref):
    R = pr_ref.shape[1]
    N = pa_ref.shape[1]
    pr = pr_ref[0]
    pa = pa_ref[0]
    sqa = sqa_ref[0]
    sqr = (pr[:, 0] * pr[:, 0] + pr[:, 1] * pr[:, 1]) + pr[:, 2] * pr[:, 2]
    dot = jax.lax.dot_general(pr.astype(jnp.bfloat16), pa.astype(jnp.bfloat16),
                              (((1,), (1,)), ((), ())),
                              preferred_element_type=jnp.float32)
    d2 = sqr[:, None] + sqa - 2.0 * dot
    d = jnp.sqrt(jnp.maximum(d2, 0.0))
    iota = jax.lax.broadcasted_iota(jnp.int32, (R, N), 1)
    cols = []
    for _ in range(K):
        m = jnp.max(d, axis=1, keepdims=True)
        sel = jnp.where(d == m, iota, N)
        j = jnp.min(sel, axis=1, keepdims=True)
        cols.append(j)
        d = jnp.where(iota == j, _NEG, d)
    idx_ref[0] = jnp.concatenate(cols, axis=1)


def _knn_idx_pallas(p):
    B, N, _ = p.shape
    R = 512
    sq = jnp.sum(p * p, axis=-1)
    return pl.pallas_call(
        _knn_body,
        grid=(B, N // R),
        in_specs=[pl.BlockSpec((1, R, 3), lambda b, r: (b, r, 0)),
                  pl.BlockSpec((1, N, 3), lambda b, r: (b, 0, 0)),
                  pl.BlockSpec((1, 1, N), lambda b, r: (b, 0, 0))],
        out_specs=pl.BlockSpec((1, R, K), lambda b, r: (b, r, 0)),
        out_shape=jax.ShapeDtypeStruct((B, N, K), jnp.int32),
    )(p, p, sq[:, None, :])


def _mm(x, W, b):
    y = jax.lax.dot_general(x.astype(jnp.bfloat16), W.astype(jnp.bfloat16),
                            (((1,), (1,)), ((), ())),
                            preferred_element_type=jnp.float32)
    return y + b


def _bn_apply(z, stats, n, gamma, beta):
    mu = stats[0:1, :] / n
    var = stats[1:2, :] / n - mu * mu
    return gamma * (z - mu) / jnp.sqrt(var + EPS) + beta


def _acc_stats(ref, z, step):
    @pl.when(step == 0)
    def _():
        ref[...] = jnp.zeros_like(ref)
    s = jnp.sum(z, axis=0)
    s2 = jnp.sum(z * z, axis=0)
    ref[0, :] += s
    ref[1, :] += s2


def _a_body(f_ref, W0_ref, b0_ref, z0_ref, st0_ref):
    z0 = _mm(f_ref[...], W0_ref[...], b0_ref[0])
    z0_ref[...] = z0
    _acc_stats(st0_ref, z0, pl.program_id(0))


def _kernel_a(f2d, W0, b0):
    M = f2d.shape[0]
    R = 2048
    return pl.pallas_call(
        _a_body,
        grid=(M // R,),
        in_specs=[pl.BlockSpec((R, 16), lambda i: (i, 0)),
                  pl.BlockSpec(W0.shape, lambda i: (0, 0)),
                  pl.BlockSpec((1, 16), lambda i: (0, 0))],
        out_specs=[pl.BlockSpec((R, 16), lambda i: (i, 0)),
                   pl.BlockSpec((8, 16), lambda i: (0, 0))],
        out_shape=[jax.ShapeDtypeStruct((M, 16), jnp.float32),
                   jax.ShapeDtypeStruct((8, 16), jnp.float32)],
    )(f2d, W0, b0[None, :])


def _b_body(p_ref, z0_ref, st0_ref, g0_ref, e0_ref, n_const, T_ref):
    f0 = jax.nn.relu(_bn_apply(z0_ref[...], st0_ref[...], n_const,
                               g0_ref[0], e0_ref[0]))
    T_ref[:, 0:3] = p_ref[...]
    T_ref[:, 3:19] = f0
    T_ref[:, 19:32] = jnp.zeros_like(T_ref[:, 19:32])


def _kernel_b(p2d, z0, st0):
    M = p2d.shape[0]
    R = 2048

    def body(p_ref, z0_ref, st0_ref, g0_ref, e0_ref, T_ref):
        _b_body(p_ref, z0_ref, st0_ref, g0_ref, e0_ref, float(M), T_ref)

    return body, M, R


def _build_table(p2d, z0, st0, g0, e0):
    body, M, R = _kernel_b(p2d, z0, st0)
    return pl.pallas_call(
        body,
        grid=(M // R,),
        in_specs=[pl.BlockSpec((R, 3), lambda i: (i, 0)),
                  pl.BlockSpec((R, 16), lambda i: (i, 0)),
                  pl.BlockSpec((8, 16), lambda i: (0, 0)),
                  pl.BlockSpec((1, 16), lambda i: (0, 0)),
                  pl.BlockSpec((1, 16), lambda i: (0, 0))],
        out_specs=pl.BlockSpec((R, 32), lambda i: (i, 0)),
        out_shape=jax.ShapeDtypeStruct((M, 32), jnp.float32),
    )(p2d, z0, st0, g0[None, :], e0[None, :])


def _chain_stage1(gc, gn, prm):
    p_c = gc[:, 0:3]
    p_n = gn[:, 0:3]
    f_c = gc[:, 3:19]
    f_n = gn[:, 3:19]
    lgc = jnp.concatenate([p_c, p_n - p_c], axis=1)
    lsc = jnp.concatenate([f_c, f_n - f_c], axis=1)
    z1 = _mm(lsc, prm['W1'], prm['b1'])
    return lgc, lsc, p_n, f_n, z1


def _chain_stage2(gc, gn, prm, st1, n):
    lgc, lsc, p_n, f_n, z1 = _chain_stage1(gc, gn, prm)
    p_off = jax.nn.relu(_bn_apply(z1, st1, n, prm['g1'], prm['e1']))
    p_tilde = p_off + p_n
    lgc_aug = jnp.concatenate([lgc, p_tilde], axis=1)
    return lsc, f_n, p_tilde, lgc_aug


def _chain_stage3(gc, gn, prm, st1, st2, n):
    lsc, f_n, p_tilde, lgc_aug = _chain_stage2(gc, gn, prm, st1, n)
    z2 = _mm(lgc_aug, prm['W2'], prm['b2'])
    f_off = jax.nn.relu(_bn_apply(z2, st2, n, prm['g2'], prm['e2']))
    f_tilde = f_off + f_n
    lsc_aug = jnp.concatenate([lsc, f_tilde], axis=1)
    return p_tilde, lgc_aug, lsc_aug


def _p1_body(gc_ref, gn_ref, W1_ref, b1_ref, st1_ref):
    prm = {'W1': W1_ref[...], 'b1': b1_ref[0]}
    _, _, _, _, z1 = _chain_stage1(gc_ref[...], gn_ref[...], prm)
    _acc_stats(st1_ref, z1, pl.program_id(0))


def _p2_body(gc_ref, gn_ref, W1_ref, b1_ref, g1_ref, e1_ref,
             W2_ref, b2_ref, W3_ref, b3_ref, st1_ref, n_const,
             pt_ref, st2_ref, st3_ref):
    prm = {'W1': W1_ref[...], 'b1': b1_ref[0], 'g1': g1_ref[0], 'e1': e1_ref[0]}
    _, _, p_tilde, lgc_aug = _chain_stage2(gc_ref[...], gn_ref[...], prm,
                                           st1_ref[...], n_const)
    pt_ref[...] = p_tilde
    z2 = _mm(lgc_aug, W2_ref[...], b2_ref[0])
    z3 = _mm(lgc_aug, W3_ref[...], b3_ref[0])
    _acc_stats(st2_ref, z2, pl.program_id(0))
    _acc_stats(st3_ref, z3, pl.program_id(0))


def _p3_body(gc_ref, gn_ref, W1_ref, b1_ref, g1_ref, e1_ref,
             W2_ref, b2_ref, g2_ref, e2_ref, W4_ref, b4_ref,
             st1_ref, st2_ref, n_const, st4_ref):
    prm = {'W1': W1_ref[...], 'b1': b1_ref[0], 'g1': g1_ref[0], 'e1': e1_ref[0],
           'W2': W2_ref[...], 'b2': b2_ref[0], 'g2': g2_ref[0], 'e2': e2_ref[0]}
    _, _, lsc_aug = _chain_stage3(gc_ref[...], gn_ref[...], prm,
                                  st1_ref[...], st2_ref[...], n_const)
    z4 = _mm(lsc_aug, W4_ref[...], b4_ref[0])
    _acc_stats(st4_ref, z4, pl.program_id(0))


def _p4_body(gc_ref, gn_ref, W1_ref, b1_ref, g1_ref, e1_ref,
             W2_ref, b2_ref, g2_ref, e2_ref, W3_ref, b3_ref, g3_ref, e3_ref,
             W4_ref, b4_ref, g4_ref, e4_ref, Wm0_ref, bm0_ref,
             Wm1_ref, bm1_ref, st1_ref, st2_ref, st3_ref, st4_ref, n_const,
             mla_ref, zm1_ref, stm1_ref):
    prm = {'W1': W1_ref[...], 'b1': b1_ref[0], 'g1': g1_ref[0], 'e1': e1_ref[0],
           'W2': W2_ref[...], 'b2': b2_ref[0], 'g2': g2_ref[0], 'e2': e2_ref[0]}
    _, lgc_aug, lsc_aug = _chain_stage3(gc_ref[...], gn_ref[...], prm,
                                        st1_ref[...], st2_ref[...], n_const)
    z3 = _mm(lgc_aug, W3_ref[...], b3_ref[0])
    p_enc = jax.nn.relu(_bn_apply(z3, st3_ref[...], n_const, g3_ref[0], e3_ref[0]))
    z4 = _mm(lsc_aug, W4_ref[...], b4_ref[0])
    f_enc = jax.nn.relu(_bn_apply(z4, st4_ref[...], n_const, g4_ref[0], e4_ref[0]))
    alc = jnp.concatenate([p_enc, f_enc], axis=1)
    kw = _mm(alc, Wm0_ref[...], bm0_ref[0])
    RT = alc.shape[0]
    P = RT // K
    kw3 = kw.reshape(P, K, 32)
    alc3 = alc.reshape(P, K, 32)
    kmax = jnp.max(kw3, axis=1, keepdims=True)
    unn = jnp.exp(kw3 - kmax)
    sm = unn / jnp.sum(unn, axis=1, keepdims=True)
    ws = jnp.sum(alc3 * sm, axis=1)
    mx = jnp.max(alc3, axis=1)
    mla = jnp.concatenate([ws, mx], axis=1)
    mla_ref[...] = mla
    zm1 = _mm(mla, Wm1_ref[...], bm1_ref[0])
    zm1_ref[...] = zm1
    _acc_stats(stm1_ref, zm1, pl.program_id(0))


def _p5_body(zm1_ref, gm1_ref, em1_ref, Wm2_ref, bm2_ref, stm1_ref, n_const,
             zm2_ref, stm2_ref):
    y1 = jax.nn.relu(_bn_apply(zm1_ref[...], stm1_ref[...], n_const,
                               gm1_ref[0], em1_ref[0]))
    zm2 = _mm(y1, Wm2_ref[...], bm2_ref[0])
    zm2_ref[...] = zm2
    _acc_stats(stm2_ref, zm2, pl.program_id(0))


def _p6_body(zm2_ref, gm2_ref, em2_ref, stm2_ref, n_const, out_ref):
    y = _bn_apply(zm2_ref[...], stm2_ref[...], n_const, gm2_ref[0], em2_ref[0])
    out_ref[...] = jnp.where(y > 0, y, 0.2 * y)


def _full(pspec):
    return pl.BlockSpec(pspec, lambda i: tuple(0 for _ in pspec))


def kernel(p, f, params):
    B, N, _ = p.shape
    M = B * N
    MK = M * K
    RT = 4096
    nP = MK // RT

    idx = _knn_idx_pallas(p)

    p2d = p.reshape(M, 3)
    f2d = f.reshape(M, 16)
    z0, st0 = _kernel_a(f2d, params['W0'], params['b0'])
    T = _build_table(p2d, z0, st0, params['g0'], params['e0'])

    idx_g = (idx + (jnp.arange(B, dtype=idx.dtype) * N)[:, None, None]).reshape(-1)
    gn = jnp.take(T, idx_g, axis=0)
    gc = jnp.repeat(T, K, axis=0)

    nBN = float(MK)
    row = lambda i: (i, 0)
    w = lambda a: pl.BlockSpec(a.shape, lambda i: (0, 0))
    v = lambda a: pl.BlockSpec((1, a.shape[0]), lambda i: (0, 0))
    st_spec = lambda c: pl.BlockSpec((8, c), lambda i: (0, 0))
    prm = params

    st1 = pl.pallas_call(
        _p1_body,
        grid=(nP,),
        in_specs=[pl.BlockSpec((RT, 32), row), pl.BlockSpec((RT, 32), row),
                  w(prm['W1']), v(prm['b1'])],
        out_specs=st_spec(3),
        out_shape=jax.ShapeDtypeStruct((8, 3), jnp.float32),
    )(gc, gn, prm['W1'], prm['b1'][None, :])

    def p2_body(*a):
        _p2_body(*a[:10], a[10], nBN, *a[11:])

    p_tilde, st2, st3 = pl.pallas_call(
        lambda gc_r, gn_r, W1_r, b1_r, g1_r, e1_r, W2_r, b2_r, W3_r, b3_r, st1_r,
               pt_r, st2_r, st3_r:
            _p2_body(gc_r, gn_r, W1_r, b1_r, g1_r, e1_r, W2_r, b2_r, W3_r, b3_r,
                     st1_r, nBN, pt_r, st2_r, st3_r),
        grid=(nP,),
        in_specs=[pl.BlockSpec((RT, 32), row), pl.BlockSpec((RT, 32), row),
                  w(prm['W1']), v(prm['b1']), v(prm['g1']), v(prm['e1']),
                  w(prm['W2']), v(prm['b2']), w(prm['W3']), v(prm['b3']),
                  st_spec(3)],
        out_specs=[pl.BlockSpec((RT, 3), row), st_spec(16), st_spec(16)],
        out_shape=[jax.ShapeDtypeStruct((MK, 3), jnp.float32),
                   jax.ShapeDtypeStruct((8, 16), jnp.float32),
                   jax.ShapeDtypeStruct((8, 16), jnp.float32)],
    )(gc, gn, prm['W1'], prm['b1'][None, :], prm['g1'][None, :], prm['e1'][None, :],
      prm['W2'], prm['b2'][None, :], prm['W3'], prm['b3'][None, :], st1)

    st4 = pl.pallas_call(
        lambda gc_r, gn_r, W1_r, b1_r, g1_r, e1_r, W2_r, b2_r, g2_r, e2_r,
               W4_r, b4_r, st1_r, st2_r, st4_r:
            _p3_body(gc_r, gn_r, W1_r, b1_r, g1_r, e1_r, W2_r, b2_r, g2_r, e2_r,
                     W4_r, b4_r, st1_r, st2_r, nBN, st4_r),
        grid=(nP,),
        in_specs=[pl.BlockSpec((RT, 32), row), pl.BlockSpec((RT, 32), row),
                  w(prm['W1']), v(prm['b1']), v(prm['g1']), v(prm['e1']),
                  w(prm['W2']), v(prm['b2']), v(prm['g2']), v(prm['e2']),
                  w(prm['W4']), v(prm['b4']), st_spec(3), st_spec(16)],
        out_specs=st_spec(16),
        out_shape=jax.ShapeDtypeStruct((8, 16), jnp.float32),
    )(gc, gn, prm['W1'], prm['b1'][None, :], prm['g1'][None, :], prm['e1'][None, :],
      prm['W2'], prm['b2'][None, :], prm['g2'][None, :], prm['e2'][None, :],
      prm['W4'], prm['b4'][None, :], st1, st2)

    mla_in, zm1, stm1 = pl.pallas_call(
        lambda gc_r, gn_r, W1_r, b1_r, g1_r, e1_r, W2_r, b2_r, g2_r, e2_r,
               W3_r, b3_r, g3_r, e3_r, W4_r, b4_r, g4_r, e4_r, Wm0_r, bm0_r,
               Wm1_r, bm1_r, st1_r, st2_r, st3_r, st4_r, mla_r, zm1_r, stm1_r:
            _p4_body(gc_r, gn_r, W1_r, b1_r, g1_r, e1_r, W2_r, b2_r, g2_r, e2_r,
                     W3_r, b3_r, g3_r, e3_r, W4_r, b4_r, g4_r, e4_r, Wm0_r, bm0_r,
                     Wm1_r, bm1_r, st1_r, st2_r, st3_r, st4_r, nBN,
                     mla_r, zm1_r, stm1_r),
        grid=(nP,),
        in_specs=[pl.BlockSpec((RT, 32), row), pl.BlockSpec((RT, 32), row),
                  w(prm['W1']), v(prm['b1']), v(prm['g1']), v(prm['e1']),
                  w(prm['W2']), v(prm['b2']), v(prm['g2']), v(prm['e2']),
                  w(prm['W3']), v(prm['b3']), v(prm['g3']), v(prm['e3']),
                  w(prm['W4']), v(prm['b4']), v(prm['g4']), v(prm['e4']),
                  w(prm['Wm0']), v(prm['bm0']), w(prm['Wm1']), v(prm['bm1']),
                  st_spec(3), st_spec(16), st_spec(16), st_spec(16)],
        out_specs=[pl.BlockSpec((RT // K, 64), row), pl.BlockSpec((RT // K, 32), row),
                   st_spec(32)],
        out_shape=[jax.ShapeDtypeStruct((M, 64), jnp.float32),
                   jax.ShapeDtypeStruct((M, 32), jnp.float32),
                   jax.ShapeDtypeStruct((8, 32), jnp.float32)],
    )(gc, gn, prm['W1'], prm['b1'][None, :], prm['g1'][None, :], prm['e1'][None, :],
      prm['W2'], prm['b2'][None, :], prm['g2'][None, :], prm['e2'][None, :],
      prm['W3'], prm['b3'][None, :], prm['g3'][None, :], prm['e3'][None, :],
      prm['W4'], prm['b4'][None, :], prm['g4'][None, :], prm['e4'][None, :],
      prm['Wm0'], prm['bm0'][None, :], prm['Wm1'], prm['bm1'][None, :],
      st1, st2, st3, st4)

    nM = float(M)
    RM = 2048
    zm2, stm2 = pl.pallas_call(
        lambda zm1_r, gm1_r, em1_r, Wm2_r, bm2_r, stm1_r, zm2_r, stm2_r:
            _p5_body(zm1_r, gm1_r, em1_r, Wm2_r, bm2_r, stm1_r, nM, zm2_r, stm2_r),
        grid=(M // RM,),
        in_specs=[pl.BlockSpec((RM, 32), row), v(prm['gm1']), v(prm['em1']),
                  w(prm['Wm2']), v(prm['bm2']), st_spec(32)],
        out_specs=[pl.BlockSpec((RM, 64), row), st_spec(64)],
        out_shape=[jax.ShapeDtypeStruct((M, 64), jnp.float32),
                   jax.ShapeDtypeStruct((8, 64), jnp.float32)],
    )(zm1, prm['gm1'][None, :], prm['em1'][None, :], prm['Wm2'], prm['bm2'][None, :],
      stm1)

    out = pl.pallas_call(
        lambda zm2_r, gm2_r, em2_r, stm2_r, out_r:
            _p6_body(zm2_r, gm2_r, em2_r, stm2_r, nM, out_r),
        grid=(M // RM,),
        in_specs=[pl.BlockSpec((RM, 64), row), v(prm['gm2']), v(prm['em2']),
                  st_spec(64)],
        out_specs=pl.BlockSpec((RM, 64), row),
        out_shape=jax.ShapeDtypeStruct((M, 64), jnp.float32),
    )(zm2, prm['gm2'][None, :], prm['em2'][None, :], stm2)

    return out.reshape(B, N, 64), p_tilde.reshape(B, N, K, 3)


# speedup vs baseline: 8.3719x; 1.2233x over previous
import functools

import jax
import jax.numpy as jnp
from jax import lax
from jax.experimental import pallas as pl
from jax.experimental.pallas import tpu as pltpu
from jax.experimental.pallas import tpu_sc as plsc

EPS = 1e-5
K = 16
_NEG = -jnp.inf


def _knn_body(pr_ref, pa_ref, sqa_ref, idx_ref):
    R = pr_ref.shape[1]
    N = pa_ref.shape[1]
    pr = pr_ref[0]
    pa = pa_ref[0]
    sqa = sqa_ref[0]
    sqr = (pr[:, 0] * pr[:, 0] + pr[:, 1] * pr[:, 1]) + pr[:, 2] * pr[:, 2]
    dot = jax.lax.dot_general(pr.astype(jnp.bfloat16), pa.astype(jnp.bfloat16),
                              (((1,), (1,)), ((), ())),
                              preferred_element_type=jnp.float32)
    d2 = sqr[:, None] + sqa - 2.0 * dot
    d = jnp.sqrt(jnp.maximum(d2, 0.0))
    iota = jax.lax.broadcasted_iota(jnp.int32, (R, N), 1)
    cols = []
    for _ in range(K):
        m = jnp.max(d, axis=1, keepdims=True)
        sel = jnp.where(d == m, iota, N)
        j = jnp.min(sel, axis=1, keepdims=True)
        cols.append(j)
        d = jnp.where(iota == j, _NEG, d)
    idx_ref[0] = jnp.concatenate(cols, axis=1)


def _knn_idx_pallas(p):
    B, N, _ = p.shape
    R = 512
    sq = jnp.sum(p * p, axis=-1)
    return pl.pallas_call(
        _knn_body,
        grid=(B, N // R),
        in_specs=[pl.BlockSpec((1, R, 3), lambda b, r: (b, r, 0)),
                  pl.BlockSpec((1, N, 3), lambda b, r: (b, 0, 0)),
                  pl.BlockSpec((1, 1, N), lambda b, r: (b, 0, 0))],
        out_specs=pl.BlockSpec((1, R, K), lambda b, r: (b, r, 0)),
        out_shape=jax.ShapeDtypeStruct((B, N, K), jnp.int32),
    )(p, p, sq[:, None, :])


def _mm(x, W, b):
    y = jax.lax.dot_general(x.astype(jnp.bfloat16), W.astype(jnp.bfloat16),
                            (((1,), (1,)), ((), ())),
                            preferred_element_type=jnp.float32)
    return y + b


def _bn_apply(z, stats, n, gamma, beta):
    mu = stats[0:1, :] / n
    var = stats[1:2, :] / n - mu * mu
    return gamma * (z - mu) / jnp.sqrt(var + EPS) + beta


def _acc_stats(ref, z, step):
    @pl.when(step == 0)
    def _():
        ref[...] = jnp.zeros_like(ref)
    s = jnp.sum(z, axis=0)
    s2 = jnp.sum(z * z, axis=0)
    ref[0, :] += s
    ref[1, :] += s2


def _a_body(f_ref, W0_ref, b0_ref, z0_ref, st0_ref):
    z0 = _mm(f_ref[...], W0_ref[...], b0_ref[0])
    z0_ref[...] = z0
    _acc_stats(st0_ref, z0, pl.program_id(0))


def _kernel_a(f2d, W0, b0):
    M = f2d.shape[0]
    R = 2048
    return pl.pallas_call(
        _a_body,
        grid=(M // R,),
        in_specs=[pl.BlockSpec((R, 16), lambda i: (i, 0)),
                  pl.BlockSpec(W0.shape, lambda i: (0, 0)),
                  pl.BlockSpec((1, 16), lambda i: (0, 0))],
        out_specs=[pl.BlockSpec((R, 16), lambda i: (i, 0)),
                   pl.BlockSpec((8, 16), lambda i: (0, 0))],
        out_shape=[jax.ShapeDtypeStruct((M, 16), jnp.float32),
                   jax.ShapeDtypeStruct((8, 16), jnp.float32)],
    )(f2d, W0, b0[None, :])


def _b_body(p_ref, z0_ref, st0_ref, g0_ref, e0_ref, n_const, T_ref):
    f0 = jax.nn.relu(_bn_apply(z0_ref[...], st0_ref[...], n_const,
                               g0_ref[0], e0_ref[0]))
    T_ref[:, 0:3] = p_ref[...]
    T_ref[:, 3:19] = f0
    T_ref[:, 19:128] = jnp.zeros_like(T_ref[:, 19:128])


def _kernel_b(p2d, z0, st0):
    M = p2d.shape[0]
    R = 2048

    def body(p_ref, z0_ref, st0_ref, g0_ref, e0_ref, T_ref):
        _b_body(p_ref, z0_ref, st0_ref, g0_ref, e0_ref, float(M), T_ref)

    return body, M, R


def _build_table(p2d, z0, st0, g0, e0):
    body, M, R = _kernel_b(p2d, z0, st0)
    return pl.pallas_call(
        body,
        grid=(M // R,),
        in_specs=[pl.BlockSpec((R, 3), lambda i: (i, 0)),
                  pl.BlockSpec((R, 16), lambda i: (i, 0)),
                  pl.BlockSpec((8, 16), lambda i: (0, 0)),
                  pl.BlockSpec((1, 16), lambda i: (0, 0)),
                  pl.BlockSpec((1, 16), lambda i: (0, 0))],
        out_specs=pl.BlockSpec((R, 128), lambda i: (i, 0)),
        out_shape=jax.ShapeDtypeStruct((M, 128), jnp.float32),
    )(p2d, z0, st0, g0[None, :], e0[None, :])


def _sc_gather(T, idx_g):
    MK = idx_g.shape[0]
    NW = 32
    per_w = MK // NW
    CH = 128
    GRP = 4
    n_grp = per_w // (GRP * CH)
    mesh = plsc.VectorSubcoreMesh(core_axis_name="c", subcore_axis_name="s")

    @functools.partial(
        pl.kernel, mesh=mesh,
        out_type=jax.ShapeDtypeStruct((MK, 128), jnp.float32),
        scratch_types=[pltpu.VMEM((per_w,), jnp.int32),
                       pltpu.VMEM((GRP, CH, 128), jnp.float32),
                       pltpu.SemaphoreType.DMA,
                       pltpu.SemaphoreType.DMA],
    )
    def k(T_hbm, idx_hbm, out_hbm, idx_v, rows_v, semA, semB):
        wid = lax.axis_index("s") * 2 + lax.axis_index("c")
        base = wid * per_w
        pltpu.sync_copy(idx_hbm.at[pl.ds(base, per_w)], idx_v)

        def grp(g, _):
            gbase = g * (GRP * CH)
            gets = []
            for j in range(GRP):
                cb = gbase + j * CH
                gets.append(pltpu.async_copy(
                    T_hbm.at[idx_v.at[pl.ds(cb, CH)]], rows_v.at[j], semA))
            puts = []
            for j in range(GRP):
                gets[j].wait()
                cb = gbase + j * CH
                puts.append(pltpu.async_copy(
                    rows_v.at[j], out_hbm.at[pl.ds(base + cb, CH)], semB))
            for h in puts:
                h.wait()
            return 0

        lax.fori_loop(0, n_grp, grp, 0)

    return k(T, idx_g)


def _chain_stage1(gc, gn, prm):
    p_c = gc[:, 0:3]
    p_n = gn[:, 0:3]
    f_c = gc[:, 3:19]
    f_n = gn[:, 3:19]
    lgc = jnp.concatenate([p_c, p_n - p_c], axis=1)
    lsc = jnp.concatenate([f_c, f_n - f_c], axis=1)
    z1 = _mm(lsc, prm['W1'], prm['b1'])
    return lgc, lsc, p_n, f_n, z1


def _chain_stage2(gc, gn, prm, st1, n):
    lgc, lsc, p_n, f_n, z1 = _chain_stage1(gc, gn, prm)
    p_off = jax.nn.relu(_bn_apply(z1, st1, n, prm['g1'], prm['e1']))
    p_tilde = p_off + p_n
    lgc_aug = jnp.concatenate([lgc, p_tilde], axis=1)
    return lsc, f_n, p_tilde, lgc_aug


def _chain_stage3(gc, gn, prm, st1, st2, n):
    lsc, f_n, p_tilde, lgc_aug = _chain_stage2(gc, gn, prm, st1, n)
    z2 = _mm(lgc_aug, prm['W2'], prm['b2'])
    f_off = jax.nn.relu(_bn_apply(z2, st2, n, prm['g2'], prm['e2']))
    f_tilde = f_off + f_n
    lsc_aug = jnp.concatenate([lsc, f_tilde], axis=1)
    return p_tilde, lgc_aug, lsc_aug


def _p1_body(gc_ref, gn_ref, W1_ref, b1_ref, st1_ref):
    prm = {'W1': W1_ref[...], 'b1': b1_ref[0]}
    _, _, _, _, z1 = _chain_stage1(gc_ref[...], gn_ref[...], prm)
    _acc_stats(st1_ref, z1, pl.program_id(0))


def _p2_body(gc_ref, gn_ref, W1_ref, b1_ref, g1_ref, e1_ref,
             W2_ref, b2_ref, W3_ref, b3_ref, st1_ref, n_const,
             pt_ref, st2_ref, st3_ref):
    prm = {'W1': W1_ref[...], 'b1': b1_ref[0], 'g1': g1_ref[0], 'e1': e1_ref[0]}
    _, _, p_tilde, lgc_aug = _chain_stage2(gc_ref[...], gn_ref[...], prm,
                                           st1_ref[...], n_const)
    pt_ref[...] = p_tilde
    z2 = _mm(lgc_aug, W2_ref[...], b2_ref[0])
    z3 = _mm(lgc_aug, W3_ref[...], b3_ref[0])
    _acc_stats(st2_ref, z2, pl.program_id(0))
    _acc_stats(st3_ref, z3, pl.program_id(0))


def _p3_body(gc_ref, gn_ref, W1_ref, b1_ref, g1_ref, e1_ref,
             W2_ref, b2_ref, g2_ref, e2_ref, W4_ref, b4_ref,
             st1_ref, st2_ref, n_const, st4_ref):
    prm = {'W1': W1_ref[...], 'b1': b1_ref[0], 'g1': g1_ref[0], 'e1': e1_ref[0],
           'W2': W2_ref[...], 'b2': b2_ref[0], 'g2': g2_ref[0], 'e2': e2_ref[0]}
    _, _, lsc_aug = _chain_stage3(gc_ref[...], gn_ref[...], prm,
                                  st1_ref[...], st2_ref[...], n_const)
    z4 = _mm(lsc_aug, W4_ref[...], b4_ref[0])
    _acc_stats(st4_ref, z4, pl.program_id(0))


def _p4_body(gc_ref, gn_ref, W1_ref, b1_ref, g1_ref, e1_ref,
             W2_ref, b2_ref, g2_ref, e2_ref, W3_ref, b3_ref, g3_ref, e3_ref,
             W4_ref, b4_ref, g4_ref, e4_ref, Wm0_ref, bm0_ref,
             Wm1_ref, bm1_ref, st1_ref, st2_ref, st3_ref, st4_ref, n_const,
             mla_ref, zm1_ref, stm1_ref):
    prm = {'W1': W1_ref[...], 'b1': b1_ref[0], 'g1': g1_ref[0], 'e1': e1_ref[0],
           'W2': W2_ref[...], 'b2': b2_ref[0], 'g2': g2_ref[0], 'e2': e2_ref[0]}
    _, lgc_aug, lsc_aug = _chain_stage3(gc_ref[...], gn_ref[...], prm,
                                        st1_ref[...], st2_ref[...], n_const)
    z3 = _mm(lgc_aug, W3_ref[...], b3_ref[0])
    p_enc = jax.nn.relu(_bn_apply(z3, st3_ref[...], n_const, g3_ref[0], e3_ref[0]))
    z4 = _mm(lsc_aug, W4_ref[...], b4_ref[0])
    f_enc = jax.nn.relu(_bn_apply(z4, st4_ref[...], n_const, g4_ref[0], e4_ref[0]))
    alc = jnp.concatenate([p_enc, f_enc], axis=1)
    kw = _mm(alc, Wm0_ref[...], bm0_ref[0])
    RT = alc.shape[0]
    P = RT // K
    kw3 = kw.reshape(P, K, 32)
    alc3 = alc.reshape(P, K, 32)
    kmax = jnp.max(kw3, axis=1, keepdims=True)
    unn = jnp.exp(kw3 - kmax)
    sm = unn / jnp.sum(unn, axis=1, keepdims=True)
    ws = jnp.sum(alc3 * sm, axis=1)
    mx = jnp.max(alc3, axis=1)
    mla = jnp.concatenate([ws, mx], axis=1)
    mla_ref[...] = mla
    zm1 = _mm(mla, Wm1_ref[...], bm1_ref[0])
    zm1_ref[...] = zm1
    _acc_stats(stm1_ref, zm1, pl.program_id(0))


def _p5_body(zm1_ref, gm1_ref, em1_ref, Wm2_ref, bm2_ref, stm1_ref, n_const,
             zm2_ref, stm2_ref):
    y1 = jax.nn.relu(_bn_apply(zm1_ref[...], stm1_ref[...], n_const,
                               gm1_ref[0], em1_ref[0]))
    zm2 = _mm(y1, Wm2_ref[...], bm2_ref[0])
    zm2_ref[...] = zm2
    _acc_stats(stm2_ref, zm2, pl.program_id(0))


def _p6_body(zm2_ref, gm2_ref, em2_ref, stm2_ref, n_const, out_ref):
    y = _bn_apply(zm2_ref[...], stm2_ref[...], n_const, gm2_ref[0], em2_ref[0])
    out_ref[...] = jnp.where(y > 0, y, 0.2 * y)


def _full(pspec):
    return pl.BlockSpec(pspec, lambda i: tuple(0 for _ in pspec))


def kernel(p, f, params):
    B, N, _ = p.shape
    M = B * N
    MK = M * K
    RT = 4096
    nP = MK // RT

    idx = _knn_idx_pallas(p)

    p2d = p.reshape(M, 3)
    f2d = f.reshape(M, 16)
    z0, st0 = _kernel_a(f2d, params['W0'], params['b0'])
    T = _build_table(p2d, z0, st0, params['g0'], params['e0'])

    idx_g = (idx + (jnp.arange(B, dtype=idx.dtype) * N)[:, None, None]).reshape(-1)
    gn = _sc_gather(T, idx_g)
    gc = jnp.repeat(T[:, :32], K, axis=0)

    nBN = float(MK)
    row = lambda i: (i, 0)
    w = lambda a: pl.BlockSpec(a.shape, lambda i: (0, 0))
    v = lambda a: pl.BlockSpec((1, a.shape[0]), lambda i: (0, 0))
    st_spec = lambda c: pl.BlockSpec((8, c), lambda i: (0, 0))
    prm = params

    st1 = pl.pallas_call(
        _p1_body,
        grid=(nP,),
        in_specs=[pl.BlockSpec((RT, 32), row), pl.BlockSpec((RT, 128), row),
                  w(prm['W1']), v(prm['b1'])],
        out_specs=st_spec(3),
        out_shape=jax.ShapeDtypeStruct((8, 3), jnp.float32),
    )(gc, gn, prm['W1'], prm['b1'][None, :])

    def p2_body(*a):
        _p2_body(*a[:10], a[10], nBN, *a[11:])

    p_tilde, st2, st3 = pl.pallas_call(
        lambda gc_r, gn_r, W1_r, b1_r, g1_r, e1_r, W2_r, b2_r, W3_r, b3_r, st1_r,
               pt_r, st2_r, st3_r:
            _p2_body(gc_r, gn_r, W1_r, b1_r, g1_r, e1_r, W2_r, b2_r, W3_r, b3_r,
                     st1_r, nBN, pt_r, st2_r, st3_r),
        grid=(nP,),
        in_specs=[pl.BlockSpec((RT, 32), row), pl.BlockSpec((RT, 128), row),
                  w(prm['W1']), v(prm['b1']), v(prm['g1']), v(prm['e1']),
                  w(prm['W2']), v(prm['b2']), w(prm['W3']), v(prm['b3']),
                  st_spec(3)],
        out_specs=[pl.BlockSpec((RT, 3), row), st_spec(16), st_spec(16)],
        out_shape=[jax.ShapeDtypeStruct((MK, 3), jnp.float32),
                   jax.ShapeDtypeStruct((8, 16), jnp.float32),
                   jax.ShapeDtypeStruct((8, 16), jnp.float32)],
    )(gc, gn, prm['W1'], prm['b1'][None, :], prm['g1'][None, :], prm['e1'][None, :],
      prm['W2'], prm['b2'][None, :], prm['W3'], prm['b3'][None, :], st1)

    st4 = pl.pallas_call(
        lambda gc_r, gn_r, W1_r, b1_r, g1_r, e1_r, W2_r, b2_r, g2_r, e2_r,
               W4_r, b4_r, st1_r, st2_r, st4_r:
            _p3_body(gc_r, gn_r, W1_r, b1_r, g1_r, e1_r, W2_r, b2_r, g2_r, e2_r,
                     W4_r, b4_r, st1_r, st2_r, nBN, st4_r),
        grid=(nP,),
        in_specs=[pl.BlockSpec((RT, 32), row), pl.BlockSpec((RT, 128), row),
                  w(prm['W1']), v(prm['b1']), v(prm['g1']), v(prm['e1']),
                  w(prm['W2']), v(prm['b2']), v(prm['g2']), v(prm['e2']),
                  w(prm['W4']), v(prm['b4']), st_spec(3), st_spec(16)],
        out_specs=st_spec(16),
        out_shape=jax.ShapeDtypeStruct((8, 16), jnp.float32),
    )(gc, gn, prm['W1'], prm['b1'][None, :], prm['g1'][None, :], prm['e1'][None, :],
      prm['W2'], prm['b2'][None, :], prm['g2'][None, :], prm['e2'][None, :],
      prm['W4'], prm['b4'][None, :], st1, st2)

    mla_in, zm1, stm1 = pl.pallas_call(
        lambda gc_r, gn_r, W1_r, b1_r, g1_r, e1_r, W2_r, b2_r, g2_r, e2_r,
               W3_r, b3_r, g3_r, e3_r, W4_r, b4_r, g4_r, e4_r, Wm0_r, bm0_r,
               Wm1_r, bm1_r, st1_r, st2_r, st3_r, st4_r, mla_r, zm1_r, stm1_r:
            _p4_body(gc_r, gn_r, W1_r, b1_r, g1_r, e1_r, W2_r, b2_r, g2_r, e2_r,
                     W3_r, b3_r, g3_r, e3_r, W4_r, b4_r, g4_r, e4_r, Wm0_r, bm0_r,
                     Wm1_r, bm1_r, st1_r, st2_r, st3_r, st4_r, nBN,
                     mla_r, zm1_r, stm1_r),
        grid=(nP,),
        in_specs=[pl.BlockSpec((RT, 32), row), pl.BlockSpec((RT, 128), row),
                  w(prm['W1']), v(prm['b1']), v(prm['g1']), v(prm['e1']),
                  w(prm['W2']), v(prm['b2']), v(prm['g2']), v(prm['e2']),
                  w(prm['W3']), v(prm['b3']), v(prm['g3']), v(prm['e3']),
                  w(prm['W4']), v(prm['b4']), v(prm['g4']), v(prm['e4']),
                  w(prm['Wm0']), v(prm['bm0']), w(prm['Wm1']), v(prm['bm1']),
                  st_spec(3), st_spec(16), st_spec(16), st_spec(16)],
        out_specs=[pl.BlockSpec((RT // K, 64), row), pl.BlockSpec((RT // K, 32), row),
                   st_spec(32)],
        out_shape=[jax.ShapeDtypeStruct((M, 64), jnp.float32),
                   jax.ShapeDtypeStruct((M, 32), jnp.float32),
                   jax.ShapeDtypeStruct((8, 32), jnp.float32)],
    )(gc, gn, prm['W1'], prm['b1'][None, :], prm['g1'][None, :], prm['e1'][None, :],
      prm['W2'], prm['b2'][None, :], prm['g2'][None, :], prm['e2'][None, :],
      prm['W3'], prm['b3'][None, :], prm['g3'][None, :], prm['e3'][None, :],
      prm['W4'], prm['b4'][None, :], prm['g4'][None, :], prm['e4'][None, :],
      prm['Wm0'], prm['bm0'][None, :], prm['Wm1'], prm['bm1'][None, :],
      st1, st2, st3, st4)

    nM = float(M)
    RM = 2048
    zm2, stm2 = pl.pallas_call(
        lambda zm1_r, gm1_r, em1_r, Wm2_r, bm2_r, stm1_r, zm2_r, stm2_r:
            _p5_body(zm1_r, gm1_r, em1_r, Wm2_r, bm2_r, stm1_r, nM, zm2_r, stm2_r),
        grid=(M // RM,),
        in_specs=[pl.BlockSpec((RM, 32), row), v(prm['gm1']), v(prm['em1']),
                  w(prm['Wm2']), v(prm['bm2']), st_spec(32)],
        out_specs=[pl.BlockSpec((RM, 64), row), st_spec(64)],
        out_shape=[jax.ShapeDtypeStruct((M, 64), jnp.float32),
                   jax.ShapeDtypeStruct((8, 64), jnp.float32)],
    )(zm1, prm['gm1'][None, :], prm['em1'][None, :], prm['Wm2'], prm['bm2'][None, :],
      stm1)

    out = pl.pallas_call(
        lambda zm2_r, gm2_r, em2_r, stm2_r, out_r:
            _p6_body(zm2_r, gm2_r, em2_r, stm2_r, nM, out_r),
        grid=(M // RM,),
        in_specs=[pl.BlockSpec((RM, 64), row), v(prm['gm2']), v(prm['em2']),
                  st_spec(64)],
        out_specs=pl.BlockSpec((RM, 64), row),
        out_shape=jax.ShapeDtypeStruct((M, 64), jnp.float32),
    )(zm2, prm['gm2'][None, :], prm['em2'][None, :], stm2)

    return out.reshape(B, N, 64), p_tilde.reshape(B, N, K, 3)


# speedup vs baseline: 8.8173x; 1.0532x over previous
import functools

import jax
import jax.numpy as jnp
from jax import lax
from jax.experimental import pallas as pl
from jax.experimental.pallas import tpu as pltpu
from jax.experimental.pallas import tpu_sc as plsc

EPS = 1e-5
K = 16
_NEG = -jnp.inf


def _knn_body(pr_ref, pa_ref, sqa_ref, idx_ref):
    R = pr_ref.shape[1]
    N = pa_ref.shape[1]
    pr = pr_ref[0]
    pa = pa_ref[0]
    sqa = sqa_ref[0]
    sqr = (pr[:, 0] * pr[:, 0] + pr[:, 1] * pr[:, 1]) + pr[:, 2] * pr[:, 2]
    dot = jax.lax.dot_general(pr.astype(jnp.bfloat16), pa.astype(jnp.bfloat16),
                              (((1,), (1,)), ((), ())),
                              preferred_element_type=jnp.float32)
    d2 = sqr[:, None] + sqa - 2.0 * dot
    d = jnp.sqrt(jnp.maximum(d2, 0.0))
    iota = jax.lax.broadcasted_iota(jnp.int32, (R, N), 1)
    cols = []
    for _ in range(K):
        m = jnp.max(d, axis=1, keepdims=True)
        sel = jnp.where(d == m, iota, N)
        j = jnp.min(sel, axis=1, keepdims=True)
        cols.append(j)
        d = jnp.where(iota == j, _NEG, d)
    idx_ref[0] = jnp.concatenate(cols, axis=1)


def _knn_idx_pallas(p):
    B, N, _ = p.shape
    R = 1024
    sq = jnp.sum(p * p, axis=-1)
    return pl.pallas_call(
        _knn_body,
        grid=(B, N // R),
        in_specs=[pl.BlockSpec((1, R, 3), lambda b, r: (b, r, 0)),
                  pl.BlockSpec((1, N, 3), lambda b, r: (b, 0, 0)),
                  pl.BlockSpec((1, 1, N), lambda b, r: (b, 0, 0))],
        out_specs=pl.BlockSpec((1, R, K), lambda b, r: (b, r, 0)),
        out_shape=jax.ShapeDtypeStruct((B, N, K), jnp.int32),
    )(p, p, sq[:, None, :])


def _mm(x, W, b):
    y = jax.lax.dot_general(x.astype(jnp.bfloat16), W.astype(jnp.bfloat16),
                            (((1,), (1,)), ((), ())),
                            preferred_element_type=jnp.float32)
    return y + b


def _bn_apply(z, stats, n, gamma, beta):
    mu = stats[0:1, :] / n
    var = stats[1:2, :] / n - mu * mu
    return gamma * (z - mu) / jnp.sqrt(var + EPS) + beta


def _acc_stats(ref, z, step):
    @pl.when(step == 0)
    def _():
        ref[...] = jnp.zeros_like(ref)
    s = jnp.sum(z, axis=0)
    s2 = jnp.sum(z * z, axis=0)
    ref[0, :] += s
    ref[1, :] += s2


def _a_body(f_ref, W0_ref, b0_ref, z0_ref, st0_ref):
    z0 = _mm(f_ref[...], W0_ref[...], b0_ref[0])
    z0_ref[...] = z0
    _acc_stats(st0_ref, z0, pl.program_id(0))


def _kernel_a(f2d, W0, b0):
    M = f2d.shape[0]
    R = 2048
    return pl.pallas_call(
        _a_body,
        grid=(M // R,),
        in_specs=[pl.BlockSpec((R, 16), lambda i: (i, 0)),
                  pl.BlockSpec(W0.shape, lambda i: (0, 0)),
                  pl.BlockSpec((1, 16), lambda i: (0, 0))],
        out_specs=[pl.BlockSpec((R, 16), lambda i: (i, 0)),
                   pl.BlockSpec((8, 16), lambda i: (0, 0))],
        out_shape=[jax.ShapeDtypeStruct((M, 16), jnp.float32),
                   jax.ShapeDtypeStruct((8, 16), jnp.float32)],
    )(f2d, W0, b0[None, :])


def _b_body(p_ref, z0_ref, st0_ref, g0_ref, e0_ref, n_const, T_ref):
    f0 = jax.nn.relu(_bn_apply(z0_ref[...], st0_ref[...], n_const,
                               g0_ref[0], e0_ref[0]))
    T_ref[:, 0:3] = p_ref[...]
    T_ref[:, 3:19] = f0
    T_ref[:, 19:128] = jnp.zeros_like(T_ref[:, 19:128])


def _kernel_b(p2d, z0, st0):
    M = p2d.shape[0]
    R = 2048

    def body(p_ref, z0_ref, st0_ref, g0_ref, e0_ref, T_ref):
        _b_body(p_ref, z0_ref, st0_ref, g0_ref, e0_ref, float(M), T_ref)

    return body, M, R


def _build_table(p2d, z0, st0, g0, e0):
    body, M, R = _kernel_b(p2d, z0, st0)
    return pl.pallas_call(
        body,
        grid=(M // R,),
        in_specs=[pl.BlockSpec((R, 3), lambda i: (i, 0)),
                  pl.BlockSpec((R, 16), lambda i: (i, 0)),
                  pl.BlockSpec((8, 16), lambda i: (0, 0)),
                  pl.BlockSpec((1, 16), lambda i: (0, 0)),
                  pl.BlockSpec((1, 16), lambda i: (0, 0))],
        out_specs=pl.BlockSpec((R, 128), lambda i: (i, 0)),
        out_shape=jax.ShapeDtypeStruct((M, 128), jnp.float32),
    )(p2d, z0, st0, g0[None, :], e0[None, :])


def _sc_gather(T, idx_g):
    MK = idx_g.shape[0]
    NW = 32
    per_w = MK // NW
    CH = 128
    GRP = 4
    n_grp = per_w // (GRP * CH)
    mesh = plsc.VectorSubcoreMesh(core_axis_name="c", subcore_axis_name="s")

    @functools.partial(
        pl.kernel, mesh=mesh,
        out_type=jax.ShapeDtypeStruct((MK, 128), jnp.float32),
        scratch_types=[pltpu.VMEM((per_w,), jnp.int32),
                       pltpu.VMEM((GRP, CH, 128), jnp.float32),
                       pltpu.SemaphoreType.DMA,
                       pltpu.SemaphoreType.DMA],
    )
    def k(T_hbm, idx_hbm, out_hbm, idx_v, rows_v, semA, semB):
        wid = lax.axis_index("s") * 2 + lax.axis_index("c")
        base = wid * per_w
        pltpu.sync_copy(idx_hbm.at[pl.ds(base, per_w)], idx_v)

        def grp(g, _):
            gbase = g * (GRP * CH)
            gets = []
            for j in range(GRP):
                cb = gbase + j * CH
                gets.append(pltpu.async_copy(
                    T_hbm.at[idx_v.at[pl.ds(cb, CH)]], rows_v.at[j], semA))
            puts = []
            for j in range(GRP):
                gets[j].wait()
                cb = gbase + j * CH
                puts.append(pltpu.async_copy(
                    rows_v.at[j], out_hbm.at[pl.ds(base + cb, CH)], semB))
            for h in puts:
                h.wait()
            return 0

        lax.fori_loop(0, n_grp, grp, 0)

    return k(T, idx_g)


def _chain_stage1(gc, gn, prm):
    p_c = gc[:, 0:3]
    p_n = gn[:, 0:3]
    f_c = gc[:, 3:19]
    f_n = gn[:, 3:19]
    lgc = jnp.concatenate([p_c, p_n - p_c], axis=1)
    lsc = jnp.concatenate([f_c, f_n - f_c], axis=1)
    z1 = _mm(lsc, prm['W1'], prm['b1'])
    return lgc, lsc, p_n, f_n, z1


def _chain_stage2(gc, gn, prm, st1, n):
    lgc, lsc, p_n, f_n, z1 = _chain_stage1(gc, gn, prm)
    p_off = jax.nn.relu(_bn_apply(z1, st1, n, prm['g1'], prm['e1']))
    p_tilde = p_off + p_n
    lgc_aug = jnp.concatenate([lgc, p_tilde], axis=1)
    return lsc, f_n, p_tilde, lgc_aug


def _chain_stage3(gc, gn, prm, st1, st2, n):
    lsc, f_n, p_tilde, lgc_aug = _chain_stage2(gc, gn, prm, st1, n)
    z2 = _mm(lgc_aug, prm['W2'], prm['b2'])
    f_off = jax.nn.relu(_bn_apply(z2, st2, n, prm['g2'], prm['e2']))
    f_tilde = f_off + f_n
    lsc_aug = jnp.concatenate([lsc, f_tilde], axis=1)
    return p_tilde, lgc_aug, lsc_aug


def _p1_body(gc_ref, gn_ref, W1_ref, b1_ref, st1_ref):
    prm = {'W1': W1_ref[...], 'b1': b1_ref[0]}
    _, _, _, _, z1 = _chain_stage1(gc_ref[...], gn_ref[...], prm)
    _acc_stats(st1_ref, z1, pl.program_id(0))


def _p2_body(gc_ref, gn_ref, W1_ref, b1_ref, g1_ref, e1_ref,
             W2_ref, b2_ref, W3_ref, b3_ref, st1_ref, n_const,
             pt_ref, st2_ref, st3_ref):
    prm = {'W1': W1_ref[...], 'b1': b1_ref[0], 'g1': g1_ref[0], 'e1': e1_ref[0]}
    _, _, p_tilde, lgc_aug = _chain_stage2(gc_ref[...], gn_ref[...], prm,
                                           st1_ref[...], n_const)
    pt_ref[...] = p_tilde
    z2 = _mm(lgc_aug, W2_ref[...], b2_ref[0])
    z3 = _mm(lgc_aug, W3_ref[...], b3_ref[0])
    _acc_stats(st2_ref, z2, pl.program_id(0))
    _acc_stats(st3_ref, z3, pl.program_id(0))


def _p3_body(gc_ref, gn_ref, W1_ref, b1_ref, g1_ref, e1_ref,
             W2_ref, b2_ref, g2_ref, e2_ref, W4_ref, b4_ref,
             st1_ref, st2_ref, n_const, st4_ref):
    prm = {'W1': W1_ref[...], 'b1': b1_ref[0], 'g1': g1_ref[0], 'e1': e1_ref[0],
           'W2': W2_ref[...], 'b2': b2_ref[0], 'g2': g2_ref[0], 'e2': e2_ref[0]}
    _, _, lsc_aug = _chain_stage3(gc_ref[...], gn_ref[...], prm,
                                  st1_ref[...], st2_ref[...], n_const)
    z4 = _mm(lsc_aug, W4_ref[...], b4_ref[0])
    _acc_stats(st4_ref, z4, pl.program_id(0))


def _p4_body(gc_ref, gn_ref, W1_ref, b1_ref, g1_ref, e1_ref,
             W2_ref, b2_ref, g2_ref, e2_ref, W3_ref, b3_ref, g3_ref, e3_ref,
             W4_ref, b4_ref, g4_ref, e4_ref, Wm0_ref, bm0_ref,
             Wm1_ref, bm1_ref, st1_ref, st2_ref, st3_ref, st4_ref, n_const,
             mla_ref, zm1_ref, stm1_ref):
    prm = {'W1': W1_ref[...], 'b1': b1_ref[0], 'g1': g1_ref[0], 'e1': e1_ref[0],
           'W2': W2_ref[...], 'b2': b2_ref[0], 'g2': g2_ref[0], 'e2': e2_ref[0]}
    _, lgc_aug, lsc_aug = _chain_stage3(gc_ref[...], gn_ref[...], prm,
                                        st1_ref[...], st2_ref[...], n_const)
    z3 = _mm(lgc_aug, W3_ref[...], b3_ref[0])
    p_enc = jax.nn.relu(_bn_apply(z3, st3_ref[...], n_const, g3_ref[0], e3_ref[0]))
    z4 = _mm(lsc_aug, W4_ref[...], b4_ref[0])
    f_enc = jax.nn.relu(_bn_apply(z4, st4_ref[...], n_const, g4_ref[0], e4_ref[0]))
    alc = jnp.concatenate([p_enc, f_enc], axis=1)
    kw = _mm(alc, Wm0_ref[...], bm0_ref[0])
    RT = alc.shape[0]
    P = RT // K
    kw3 = kw.reshape(P, K, 32)
    alc3 = alc.reshape(P, K, 32)
    kmax = jnp.max(kw3, axis=1, keepdims=True)
    unn = jnp.exp(kw3 - kmax)
    sm = unn / jnp.sum(unn, axis=1, keepdims=True)
    ws = jnp.sum(alc3 * sm, axis=1)
    mx = jnp.max(alc3, axis=1)
    mla = jnp.concatenate([ws, mx], axis=1)
    mla_ref[...] = mla
    zm1 = _mm(mla, Wm1_ref[...], bm1_ref[0])
    zm1_ref[...] = zm1
    _acc_stats(stm1_ref, zm1, pl.program_id(0))


def _p5_body(zm1_ref, gm1_ref, em1_ref, Wm2_ref, bm2_ref, stm1_ref, n_const,
             zm2_ref, stm2_ref):
    y1 = jax.nn.relu(_bn_apply(zm1_ref[...], stm1_ref[...], n_const,
                               gm1_ref[0], em1_ref[0]))
    zm2 = _mm(y1, Wm2_ref[...], bm2_ref[0])
    zm2_ref[...] = zm2
    _acc_stats(stm2_ref, zm2, pl.program_id(0))


def _p6_body(zm2_ref, gm2_ref, em2_ref, stm2_ref, n_const, out_ref):
    y = _bn_apply(zm2_ref[...], stm2_ref[...], n_const, gm2_ref[0], em2_ref[0])
    out_ref[...] = jnp.where(y > 0, y, 0.2 * y)


def _full(pspec):
    return pl.BlockSpec(pspec, lambda i: tuple(0 for _ in pspec))


def kernel(p, f, params):
    B, N, _ = p.shape
    M = B * N
    MK = M * K
    RT = 4096
    nP = MK // RT

    idx = _knn_idx_pallas(p)

    p2d = p.reshape(M, 3)
    f2d = f.reshape(M, 16)
    z0, st0 = _kernel_a(f2d, params['W0'], params['b0'])
    T = _build_table(p2d, z0, st0, params['g0'], params['e0'])

    idx_g = (idx + (jnp.arange(B, dtype=idx.dtype) * N)[:, None, None]).reshape(-1)
    gn = _sc_gather(T, idx_g)
    gc = jnp.repeat(T[:, :32], K, axis=0)

    nBN = float(MK)
    row = lambda i: (i, 0)
    w = lambda a: pl.BlockSpec(a.shape, lambda i: (0, 0))
    v = lambda a: pl.BlockSpec((1, a.shape[0]), lambda i: (0, 0))
    st_spec = lambda c: pl.BlockSpec((8, c), lambda i: (0, 0))
    prm = params

    st1 = pl.pallas_call(
        _p1_body,
        grid=(nP,),
        in_specs=[pl.BlockSpec((RT, 32), row), pl.BlockSpec((RT, 128), row),
                  w(prm['W1']), v(prm['b1'])],
        out_specs=st_spec(3),
        out_shape=jax.ShapeDtypeStruct((8, 3), jnp.float32),
    )(gc, gn, prm['W1'], prm['b1'][None, :])

    def p2_body(*a):
        _p2_body(*a[:10], a[10], nBN, *a[11:])

    p_tilde, st2, st3 = pl.pallas_call(
        lambda gc_r, gn_r, W1_r, b1_r, g1_r, e1_r, W2_r, b2_r, W3_r, b3_r, st1_r,
               pt_r, st2_r, st3_r:
            _p2_body(gc_r, gn_r, W1_r, b1_r, g1_r, e1_r, W2_r, b2_r, W3_r, b3_r,
                     st1_r, nBN, pt_r, st2_r, st3_r),
        grid=(nP,),
        in_specs=[pl.BlockSpec((RT, 32), row), pl.BlockSpec((RT, 128), row),
                  w(prm['W1']), v(prm['b1']), v(prm['g1']), v(prm['e1']),
                  w(prm['W2']), v(prm['b2']), w(prm['W3']), v(prm['b3']),
                  st_spec(3)],
        out_specs=[pl.BlockSpec((RT, 3), row), st_spec(16), st_spec(16)],
        out_shape=[jax.ShapeDtypeStruct((MK, 3), jnp.float32),
                   jax.ShapeDtypeStruct((8, 16), jnp.float32),
                   jax.ShapeDtypeStruct((8, 16), jnp.float32)],
    )(gc, gn, prm['W1'], prm['b1'][None, :], prm['g1'][None, :], prm['e1'][None, :],
      prm['W2'], prm['b2'][None, :], prm['W3'], prm['b3'][None, :], st1)

    st4 = pl.pallas_call(
        lambda gc_r, gn_r, W1_r, b1_r, g1_r, e1_r, W2_r, b2_r, g2_r, e2_r,
               W4_r, b4_r, st1_r, st2_r, st4_r:
            _p3_body(gc_r, gn_r, W1_r, b1_r, g1_r, e1_r, W2_r, b2_r, g2_r, e2_r,
                     W4_r, b4_r, st1_r, st2_r, nBN, st4_r),
        grid=(nP,),
        in_specs=[pl.BlockSpec((RT, 32), row), pl.BlockSpec((RT, 128), row),
                  w(prm['W1']), v(prm['b1']), v(prm['g1']), v(prm['e1']),
                  w(prm['W2']), v(prm['b2']), v(prm['g2']), v(prm['e2']),
                  w(prm['W4']), v(prm['b4']), st_spec(3), st_spec(16)],
        out_specs=st_spec(16),
        out_shape=jax.ShapeDtypeStruct((8, 16), jnp.float32),
    )(gc, gn, prm['W1'], prm['b1'][None, :], prm['g1'][None, :], prm['e1'][None, :],
      prm['W2'], prm['b2'][None, :], prm['g2'][None, :], prm['e2'][None, :],
      prm['W4'], prm['b4'][None, :], st1, st2)

    mla_in, zm1, stm1 = pl.pallas_call(
        lambda gc_r, gn_r, W1_r, b1_r, g1_r, e1_r, W2_r, b2_r, g2_r, e2_r,
               W3_r, b3_r, g3_r, e3_r, W4_r, b4_r, g4_r, e4_r, Wm0_r, bm0_r,
               Wm1_r, bm1_r, st1_r, st2_r, st3_r, st4_r, mla_r, zm1_r, stm1_r:
            _p4_body(gc_r, gn_r, W1_r, b1_r, g1_r, e1_r, W2_r, b2_r, g2_r, e2_r,
                     W3_r, b3_r, g3_r, e3_r, W4_r, b4_r, g4_r, e4_r, Wm0_r, bm0_r,
                     Wm1_r, bm1_r, st1_r, st2_r, st3_r, st4_r, nBN,
                     mla_r, zm1_r, stm1_r),
        grid=(nP,),
        in_specs=[pl.BlockSpec((RT, 32), row), pl.BlockSpec((RT, 128), row),
                  w(prm['W1']), v(prm['b1']), v(prm['g1']), v(prm['e1']),
                  w(prm['W2']), v(prm['b2']), v(prm['g2']), v(prm['e2']),
                  w(prm['W3']), v(prm['b3']), v(prm['g3']), v(prm['e3']),
                  w(prm['W4']), v(prm['b4']), v(prm['g4']), v(prm['e4']),
                  w(prm['Wm0']), v(prm['bm0']), w(prm['Wm1']), v(prm['bm1']),
                  st_spec(3), st_spec(16), st_spec(16), st_spec(16)],
        out_specs=[pl.BlockSpec((RT // K, 64), row), pl.BlockSpec((RT // K, 32), row),
                   st_spec(32)],
        out_shape=[jax.ShapeDtypeStruct((M, 64), jnp.float32),
                   jax.ShapeDtypeStruct((M, 32), jnp.float32),
                   jax.ShapeDtypeStruct((8, 32), jnp.float32)],
    )(gc, gn, prm['W1'], prm['b1'][None, :], prm['g1'][None, :], prm['e1'][None, :],
      prm['W2'], prm['b2'][None, :], prm['g2'][None, :], prm['e2'][None, :],
      prm['W3'], prm['b3'][None, :], prm['g3'][None, :], prm['e3'][None, :],
      prm['W4'], prm['b4'][None, :], prm['g4'][None, :], prm['e4'][None, :],
      prm['Wm0'], prm['bm0'][None, :], prm['Wm1'], prm['bm1'][None, :],
      st1, st2, st3, st4)

    nM = float(M)
    RM = 2048
    zm2, stm2 = pl.pallas_call(
        lambda zm1_r, gm1_r, em1_r, Wm2_r, bm2_r, stm1_r, zm2_r, stm2_r:
            _p5_body(zm1_r, gm1_r, em1_r, Wm2_r, bm2_r, stm1_r, nM, zm2_r, stm2_r),
        grid=(M // RM,),
        in_specs=[pl.BlockSpec((RM, 32), row), v(prm['gm1']), v(prm['em1']),
                  w(prm['Wm2']), v(prm['bm2']), st_spec(32)],
        out_specs=[pl.BlockSpec((RM, 64), row), st_spec(64)],
        out_shape=[jax.ShapeDtypeStruct((M, 64), jnp.float32),
                   jax.ShapeDtypeStruct((8, 64), jnp.float32)],
    )(zm1, prm['gm1'][None, :], prm['em1'][None, :], prm['Wm2'], prm['bm2'][None, :],
      stm1)

    out = pl.pallas_call(
        lambda zm2_r, gm2_r, em2_r, stm2_r, out_r:
            _p6_body(zm2_r, gm2_r, em2_r, stm2_r, nM, out_r),
        grid=(M // RM,),
        in_specs=[pl.BlockSpec((RM, 64), row), v(prm['gm2']), v(prm['em2']),
                  st_spec(64)],
        out_specs=pl.BlockSpec((RM, 64), row),
        out_shape=jax.ShapeDtypeStruct((M, 64), jnp.float32),
    )(zm2, prm['gm2'][None, :], prm['em2'][None, :], stm2)

    return out.reshape(B, N, 64), p_tilde.reshape(B, N, K, 3)
